# baseline (speedup 1.0000x reference)
import functools

import jax
import jax.numpy as jnp
from jax import lax
from jax.experimental import pallas as pl
from jax.experimental.pallas import tpu as pltpu
from jax.experimental.pallas import tpu_sc as plsc

Q_MIN = -128.0
Q_MAX = 127.0
_MAGIC = 1.5 * (2.0 ** 23)
_PRE = 1024.0
_TC_BLK = 16384


def _dequant_block(wt_ref, sc_ref, out_ref):
  d = wt_ref.shape[0]
  t = wt_ref[...].T
  t = jnp.minimum(jnp.maximum(t, -_PRE), _PRE)
  t = (t + _MAGIC) - _MAGIC
  t = jnp.minimum(jnp.maximum(t, Q_MIN), Q_MAX)
  t = t * sc_ref[...][:, None]
  t3 = t.reshape(t.shape[0] // 2, 2, d)
  out_ref[:, :d] = t3[:, 0, :]
  out_ref[:, d:] = t3[:, 1, :]


@functools.cache
def _build_tc(V, D):
  return pl.pallas_call(
      _dequant_block,
      grid=(pl.cdiv(V, _TC_BLK),),
      in_specs=[
          pl.BlockSpec((D, _TC_BLK), lambda j: (0, j)),
          pl.BlockSpec((_TC_BLK,), lambda j: (j,)),
      ],
      out_specs=pl.BlockSpec((_TC_BLK // 2, 2 * D), lambda j: (j, 0)),
      out_shape=jax.ShapeDtypeStruct((V // 2, 2 * D), jnp.float32),
  )


@functools.cache
def _build_sc(V, D, B):
  info = plsc.get_sparse_core_info()
  NC, NS, L = info.num_cores, info.num_subcores, info.num_lanes
  NW = NC * NS
  assert B % (8 * NW) == 0
  b_per_w = B // NW
  mesh = plsc.VectorSubcoreMesh(core_axis_name="c", subcore_axis_name="s")

  @functools.partial(
      pl.kernel,
      out_type=jax.ShapeDtypeStruct((B, 2 * D), jnp.float32),
      mesh=mesh,
      scratch_types=[
          pltpu.VMEM((b_per_w,), jnp.int32),
          pltpu.VMEM((b_per_w, 2 * D), jnp.float32),
          pltpu.SemaphoreType.DMA,
      ],
  )
  def gather_rows(table_hbm, idx_hbm, out_hbm, idx_v, rows_v, sem):
    wid = lax.axis_index("s") * NC + lax.axis_index("c")
    base = wid * b_per_w
    pltpu.sync_copy(idx_hbm.at[pl.ds(base, b_per_w)], idx_v)
    for i in range(b_per_w // L):
      idx_v[pl.ds(i * L, L)] = lax.shift_right_logical(idx_v[pl.ds(i * L, L)], 1)
    pltpu.async_copy(table_hbm.at[idx_v], rows_v, sem).wait()
    pltpu.sync_copy(rows_v, out_hbm.at[pl.ds(base, b_per_w)])

  return gather_rows


def kernel(x, weights, scales):
  V, D = weights.shape
  (B,) = x.shape
  xi = x.astype(jnp.int32)
  table = _build_tc(V, D)(weights.T, scales)
  pairs = _build_sc(V, D, B)(table, xi)
  return jnp.where((xi % 2 == 0)[:, None], pairs[:, :D], pairs[:, D:])


# speedup vs baseline: 1.8761x; 1.8761x over previous
import functools

import jax
import jax.numpy as jnp
from jax import lax
from jax.experimental import pallas as pl
from jax.experimental.pallas import tpu as pltpu
from jax.experimental.pallas import tpu_sc as plsc

Q_MIN = -128.0
Q_MAX = 127.0
_MAGIC = 1.5 * (2.0 ** 23)
_PRE = 1024.0
_TC_BLK = 16384


def _dequant_block(wt_ref, sc_ref, out_ref):
  d = wt_ref.shape[0]
  t = wt_ref[...].T
  t = jnp.minimum(jnp.maximum(t, -_PRE), _PRE)
  t = (t + _MAGIC) - _MAGIC
  t = jnp.minimum(jnp.maximum(t, Q_MIN), Q_MAX)
  t = t * sc_ref[...][:, None]
  out_ref[:, :d] = t
  out_ref[:, d:] = jnp.zeros_like(out_ref[:, d:])


@functools.cache
def _build_tc(V, D):
  return pl.pallas_call(
      _dequant_block,
      grid=(pl.cdiv(V, _TC_BLK),),
      in_specs=[
          pl.BlockSpec((D, _TC_BLK), lambda j: (0, j)),
          pl.BlockSpec((_TC_BLK,), lambda j: (j,)),
      ],
      out_specs=pl.BlockSpec((_TC_BLK, 2 * D), lambda j: (j, 0)),
      out_shape=jax.ShapeDtypeStruct((V, 2 * D), jnp.float32),
  )


@functools.cache
def _build_sc(V, D, B):
  info = plsc.get_sparse_core_info()
  NC, NS, L = info.num_cores, info.num_subcores, info.num_lanes
  NW = NC * NS
  assert B % (8 * NW) == 0
  b_per_w = B // NW
  mesh = plsc.VectorSubcoreMesh(core_axis_name="c", subcore_axis_name="s")

  @functools.partial(
      pl.kernel,
      out_type=jax.ShapeDtypeStruct((B, 2 * D), jnp.float32),
      mesh=mesh,
      scratch_types=[
          pltpu.VMEM((b_per_w,), jnp.int32),
          pltpu.VMEM((b_per_w, 2 * D), jnp.float32),
          pltpu.SemaphoreType.DMA,
      ],
  )
  def gather_rows(table_hbm, idx_hbm, out_hbm, idx_v, rows_v, sem):
    wid = lax.axis_index("s") * NC + lax.axis_index("c")
    base = wid * b_per_w
    pltpu.sync_copy(idx_hbm.at[pl.ds(base, b_per_w)], idx_v)
    pltpu.async_copy(table_hbm.at[idx_v], rows_v, sem).wait()
    pltpu.sync_copy(rows_v, out_hbm.at[pl.ds(base, b_per_w)])

  return gather_rows


def kernel(x, weights, scales):
  V, D = weights.shape
  (B,) = x.shape
  table = _build_tc(V, D)(weights.T, scales)
  rows = _build_sc(V, D, B)(table, x.astype(jnp.int32))
  return rows[:, :D]


# speedup vs baseline: 1.9523x; 1.0406x over previous
import functools

import jax
import jax.numpy as jnp
from jax import lax
from jax.experimental import pallas as pl
from jax.experimental.pallas import tpu as pltpu
from jax.experimental.pallas import tpu_sc as plsc

Q_MIN = -128.0
Q_MAX = 127.0
_MAGIC = 1.5 * (2.0 ** 23)
_PRE = 1024.0
_TC_BLK = 24576


def _dequant_block(wt_ref, sc_ref, out_ref):
  d = wt_ref.shape[0]
  t = wt_ref[...].T
  t = jnp.minimum(jnp.maximum(t, -_PRE), _PRE)
  t = (t + _MAGIC) - _MAGIC
  t = jnp.minimum(jnp.maximum(t, Q_MIN), Q_MAX)
  t = t * sc_ref[...][:, None]
  out_ref[:, :d] = t
  out_ref[:, d:] = jnp.zeros_like(out_ref[:, d:])


@functools.cache
def _build_tc(V, D):
  return pl.pallas_call(
      _dequant_block,
      grid=(pl.cdiv(V, _TC_BLK),),
      in_specs=[
          pl.BlockSpec((D, _TC_BLK), lambda j: (0, j)),
          pl.BlockSpec((_TC_BLK,), lambda j: (j,)),
      ],
      out_specs=pl.BlockSpec((_TC_BLK, 2 * D), lambda j: (j, 0)),
      out_shape=jax.ShapeDtypeStruct((V, 2 * D), jnp.float32),
  )


@functools.cache
def _build_sc(V, D, B):
  info = plsc.get_sparse_core_info()
  NC, NS, L = info.num_cores, info.num_subcores, info.num_lanes
  NW = NC * NS
  assert B % (8 * NW) == 0
  b_per_w = B // NW
  mesh = plsc.VectorSubcoreMesh(core_axis_name="c", subcore_axis_name="s")

  @functools.partial(
      pl.kernel,
      out_type=jax.ShapeDtypeStruct((B, 2 * D), jnp.float32),
      mesh=mesh,
      scratch_types=[
          pltpu.VMEM((b_per_w,), jnp.int32),
          pltpu.VMEM((b_per_w, 2 * D), jnp.float32),
          pltpu.SemaphoreType.DMA,
      ],
  )
  def gather_rows(table_hbm, idx_hbm, out_hbm, idx_v, rows_v, sem):
    wid = lax.axis_index("s") * NC + lax.axis_index("c")
    base = wid * b_per_w
    pltpu.sync_copy(idx_hbm.at[pl.ds(base, b_per_w)], idx_v)
    pltpu.async_copy(table_hbm.at[idx_v], rows_v, sem).wait()
    pltpu.sync_copy(rows_v, out_hbm.at[pl.ds(base, b_per_w)])

  return gather_rows


def kernel(x, weights, scales):
  V, D = weights.shape
  (B,) = x.shape
  table = _build_tc(V, D)(weights.T, scales)
  rows = _build_sc(V, D, B)(table, x.astype(jnp.int32))
  return rows[:, :D]
